# baseline (speedup 1.0000x reference)
import functools

import jax
import jax.numpy as jnp
from jax import lax
from jax.experimental import pallas as pl
from jax.experimental.pallas import tpu as pltpu
from jax.experimental.pallas import tpu_sc as plsc

B = 16384
VOCAB = 1000000
EMB = 64
NUM_LABELS = 5
HID = 200

_HALF = 500224
_NC = 2
_NS = 16
_NW = _NC * _NS
_B_PER_W = B // _NW

_BP = 512


def _pack_body(tl_ref, tr_ref, ident_ref, out_ref):
    ident = ident_ref[...]
    dn = (((0,), (0,)), ((), ()))
    out_ref[:, 0:EMB] = lax.dot_general(
        tl_ref[...], ident, dn, preferred_element_type=jnp.float32)
    out_ref[:, EMB:2 * EMB] = lax.dot_general(
        tr_ref[...], ident, dn, preferred_element_type=jnp.float32)


def _pack(tT, ident):
    nb = _HALF // _BP
    return pl.pallas_call(
        _pack_body,
        grid=(nb,),
        in_specs=[
            pl.BlockSpec((EMB, _BP), lambda i: (0, i)),
            pl.BlockSpec((EMB, _BP), lambda i, _nb=nb: (0, i + _nb)),
            pl.BlockSpec((EMB, EMB), lambda i: (0, 0)),
        ],
        out_specs=pl.BlockSpec((_BP, 2 * EMB), lambda i: (i, 0)),
        out_shape=jax.ShapeDtypeStruct((_HALF, 2 * EMB), jnp.float32),
    )(tT, tT, ident)


@functools.cache
def _make_sc_gather():
    mesh = plsc.VectorSubcoreMesh(core_axis_name="c", subcore_axis_name="s")

    @functools.partial(
        pl.kernel,
        mesh=mesh,
        out_type=jax.ShapeDtypeStruct((B, 2 * EMB), jnp.float32),
        scratch_types=[
            pltpu.VMEM((_B_PER_W,), jnp.int32),
            pltpu.VMEM((_B_PER_W, 2 * EMB), jnp.float32),
            pltpu.SemaphoreType.DMA,
        ],
    )
    def sc_gather(table_hbm, idx_hbm, out_hbm, idx_v, rows_v, sem):
        wid = lax.axis_index("s") * _NC + lax.axis_index("c")
        base = wid * _B_PER_W
        pltpu.sync_copy(idx_hbm.at[pl.ds(base, _B_PER_W)], idx_v)
        for v in range(_B_PER_W // 16):
            w = idx_v[pl.ds(v * 16, 16)]
            idx_v[pl.ds(v * 16, 16)] = jnp.where(w >= _HALF, w - _HALF, w)
        pltpu.async_copy(table_hbm.at[idx_v], rows_v, sem).wait()
        pltpu.sync_copy(rows_v, out_hbm.at[pl.ds(base, _B_PER_W)])

    return sc_gather


def _mlp_body(g_ref, word_ref, prev_ref, wh1_ref, wh2_ref, bh_ref, wo_ref,
              bo_ref, out_ref):
    hi = (word_ref[...] >= _HALF).astype(jnp.float32)
    x1 = g_ref[:, 0:EMB] * (1.0 - hi) + g_ref[:, EMB:2 * EMB] * hi
    h = jnp.dot(x1, wh1_ref[...], preferred_element_type=jnp.float32)
    h = h + jnp.dot(prev_ref[...], wh2_ref[...], preferred_element_type=jnp.float32)
    h = h + bh_ref[...]
    y = h * jax.nn.sigmoid(h)
    logits = jnp.dot(y, wo_ref[...], preferred_element_type=jnp.float32) + bo_ref[...]
    m = jnp.max(logits, axis=-1, keepdims=True)
    e = jnp.exp(logits - m)
    out_ref[...] = e / jnp.sum(e, axis=-1, keepdims=True)


_BB = 2048


def _mlp(g, word2d, prev, wh1, wh2, bh, wo, bo):
    return pl.pallas_call(
        _mlp_body,
        grid=(B // _BB,),
        in_specs=[
            pl.BlockSpec((_BB, 2 * EMB), lambda i: (i, 0)),
            pl.BlockSpec((_BB, 1), lambda i: (i, 0)),
            pl.BlockSpec((_BB, NUM_LABELS), lambda i: (i, 0)),
            pl.BlockSpec((EMB, HID), lambda i: (0, 0)),
            pl.BlockSpec((NUM_LABELS, HID), lambda i: (0, 0)),
            pl.BlockSpec((1, HID), lambda i: (0, 0)),
            pl.BlockSpec((HID, NUM_LABELS), lambda i: (0, 0)),
            pl.BlockSpec((1, NUM_LABELS), lambda i: (0, 0)),
        ],
        out_specs=pl.BlockSpec((_BB, NUM_LABELS), lambda i: (i, 0)),
        out_shape=jax.ShapeDtypeStruct((B, NUM_LABELS), jnp.float32),
    )(g, word2d, prev, wh1, wh2, bh, wo, bo)


@jax.jit
def kernel(word, prev_label, emb_table, W_h, b_h, W_o, b_o):
    ident = jnp.eye(EMB, dtype=jnp.float32)
    packed = _pack(emb_table.T, ident)
    g = _make_sc_gather()(packed, word)
    return _mlp(
        g,
        word.reshape(B, 1),
        prev_label,
        W_h[:EMB],
        W_h[EMB:],
        b_h.reshape(1, HID),
        W_o,
        b_o.reshape(1, NUM_LABELS),
    )


# speedup vs baseline: 2.7147x; 2.7147x over previous
import functools

import jax
import jax.numpy as jnp
from jax import lax
from jax.experimental import pallas as pl
from jax.experimental.pallas import tpu as pltpu
from jax.experimental.pallas import tpu_sc as plsc

B = 16384
VOCAB = 1000000
EMB = 64
NUM_LABELS = 5
HID = 200

_HALF = 524288
_NC = 2
_NS = 16
_NW = _NC * _NS
_B_PER_W = B // _NW

_BP = 8192


def _pack_body(tl_ref, tr_ref, ident_ref, out_ref):
    ident = ident_ref[...]
    dn = (((0,), (0,)), ((), ()))
    out_ref[:, 0:EMB] = lax.dot_general(
        tl_ref[...], ident, dn, preferred_element_type=jnp.float32)
    out_ref[:, EMB:2 * EMB] = lax.dot_general(
        tr_ref[...], ident, dn, preferred_element_type=jnp.float32)


def _pack(tT, ident):
    nb = _HALF // _BP
    last = (VOCAB - 1) // _BP

    return pl.pallas_call(
        _pack_body,
        grid=(nb,),
        in_specs=[
            pl.BlockSpec((EMB, _BP), lambda i: (0, i)),
            pl.BlockSpec(
                (EMB, _BP),
                lambda i, _nb=nb, _last=last: (0, jnp.minimum(i + _nb, _last)),
            ),
            pl.BlockSpec((EMB, EMB), lambda i: (0, 0)),
        ],
        out_specs=pl.BlockSpec((_BP, 2 * EMB), lambda i: (i, 0)),
        out_shape=jax.ShapeDtypeStruct((_HALF, 2 * EMB), jnp.float32),
    )(tT, tT, ident)


@functools.cache
def _make_sc_gather():
    mesh = plsc.VectorSubcoreMesh(core_axis_name="c", subcore_axis_name="s")

    @functools.partial(
        pl.kernel,
        mesh=mesh,
        out_type=jax.ShapeDtypeStruct((B, 2 * EMB), jnp.float32),
        scratch_types=[
            pltpu.VMEM((_B_PER_W,), jnp.int32),
            pltpu.VMEM((_B_PER_W, 2 * EMB), jnp.float32),
            pltpu.SemaphoreType.DMA,
        ],
    )
    def sc_gather(table_hbm, idx_hbm, out_hbm, idx_v, rows_v, sem):
        wid = lax.axis_index("s") * _NC + lax.axis_index("c")
        base = wid * _B_PER_W
        pltpu.sync_copy(idx_hbm.at[pl.ds(base, _B_PER_W)], idx_v)
        for v in range(_B_PER_W // 16):
            w = idx_v[pl.ds(v * 16, 16)]
            idx_v[pl.ds(v * 16, 16)] = jnp.where(w >= _HALF, w - _HALF, w)
        pltpu.async_copy(table_hbm.at[idx_v], rows_v, sem).wait()
        pltpu.sync_copy(rows_v, out_hbm.at[pl.ds(base, _B_PER_W)])

    return sc_gather


def _mlp_body(g_ref, word_ref, prev_ref, wh1_ref, wh2_ref, bh_ref, wo_ref,
              bo_ref, out_ref):
    hi = (word_ref[...] >= _HALF).astype(jnp.float32)
    x1 = g_ref[:, 0:EMB] * (1.0 - hi) + g_ref[:, EMB:2 * EMB] * hi
    h = jnp.dot(x1, wh1_ref[...], preferred_element_type=jnp.float32)
    h = h + jnp.dot(prev_ref[...], wh2_ref[...], preferred_element_type=jnp.float32)
    h = h + bh_ref[...]
    y = h * jax.nn.sigmoid(h)
    logits = jnp.dot(y, wo_ref[...], preferred_element_type=jnp.float32) + bo_ref[...]
    m = jnp.max(logits, axis=-1, keepdims=True)
    e = jnp.exp(logits - m)
    out_ref[...] = e / jnp.sum(e, axis=-1, keepdims=True)


_BB = 2048


def _mlp(g, word2d, prev, wh1, wh2, bh, wo, bo):
    return pl.pallas_call(
        _mlp_body,
        grid=(B // _BB,),
        in_specs=[
            pl.BlockSpec((_BB, 2 * EMB), lambda i: (i, 0)),
            pl.BlockSpec((_BB, 1), lambda i: (i, 0)),
            pl.BlockSpec((_BB, NUM_LABELS), lambda i: (i, 0)),
            pl.BlockSpec((EMB, HID), lambda i: (0, 0)),
            pl.BlockSpec((NUM_LABELS, HID), lambda i: (0, 0)),
            pl.BlockSpec((1, HID), lambda i: (0, 0)),
            pl.BlockSpec((HID, NUM_LABELS), lambda i: (0, 0)),
            pl.BlockSpec((1, NUM_LABELS), lambda i: (0, 0)),
        ],
        out_specs=pl.BlockSpec((_BB, NUM_LABELS), lambda i: (i, 0)),
        out_shape=jax.ShapeDtypeStruct((B, NUM_LABELS), jnp.float32),
    )(g, word2d, prev, wh1, wh2, bh, wo, bo)


@jax.jit
def kernel(word, prev_label, emb_table, W_h, b_h, W_o, b_o):
    ident = jnp.eye(EMB, dtype=jnp.float32)
    packed = _pack(emb_table.T, ident)
    g = _make_sc_gather()(packed, word)
    return _mlp(
        g,
        word.reshape(B, 1),
        prev_label,
        W_h[:EMB],
        W_h[EMB:],
        b_h.reshape(1, HID),
        W_o,
        b_o.reshape(1, NUM_LABELS),
    )


# speedup vs baseline: 3.2407x; 1.1938x over previous
import functools

import jax
import jax.numpy as jnp
from jax import lax
from jax.experimental import pallas as pl
from jax.experimental.pallas import tpu as pltpu
from jax.experimental.pallas import tpu_sc as plsc

B = 16384
VOCAB = 1000000
EMB = 64
NUM_LABELS = 5
HID = 200

_NQ = 1 << 18
_NC = 2
_NS = 16
_NW = _NC * _NS
_B_PER_W = B // _NW

_BP = 8192
_MASK_HI = -65536


def _pack_body(t0_ref, t1_ref, t2_ref, t3_ref, ident_ref, out_ref):
    ident = ident_ref[...]
    dn = (((0,), (0,)), ((), ()))

    def bits(t_ref):
        x = lax.dot_general(t_ref[...], ident, dn,
                            preferred_element_type=jnp.float32)
        x = x.astype(jnp.bfloat16).astype(jnp.float32)
        return lax.bitcast_convert_type(x, jnp.int32)

    b0, b1, b2, b3 = bits(t0_ref), bits(t1_ref), bits(t2_ref), bits(t3_ref)
    out_ref[:, 0:EMB] = lax.shift_right_logical(b0, 16) | (b1 & _MASK_HI)
    out_ref[:, EMB:2 * EMB] = lax.shift_right_logical(b2, 16) | (b3 & _MASK_HI)


def _pack(tT, ident):
    nb = _NQ // _BP
    last = (VOCAB - 1) // _BP

    def qmap(k):
        off = k * nb
        return lambda i, _off=off, _last=last: (0, jnp.minimum(i + _off, _last))

    return pl.pallas_call(
        _pack_body,
        grid=(nb,),
        in_specs=[
            pl.BlockSpec((EMB, _BP), qmap(0)),
            pl.BlockSpec((EMB, _BP), qmap(1)),
            pl.BlockSpec((EMB, _BP), qmap(2)),
            pl.BlockSpec((EMB, _BP), qmap(3)),
            pl.BlockSpec((EMB, EMB), lambda i: (0, 0)),
        ],
        out_specs=pl.BlockSpec((_BP, 2 * EMB), lambda i: (i, 0)),
        out_shape=jax.ShapeDtypeStruct((_NQ, 2 * EMB), jnp.int32),
    )(tT, tT, tT, tT, ident)


@functools.cache
def _make_sc_gather():
    mesh = plsc.VectorSubcoreMesh(core_axis_name="c", subcore_axis_name="s")

    @functools.partial(
        pl.kernel,
        mesh=mesh,
        out_type=jax.ShapeDtypeStruct((B, 2 * EMB), jnp.int32),
        scratch_types=[
            pltpu.VMEM((_B_PER_W,), jnp.int32),
            pltpu.VMEM((_B_PER_W, 2 * EMB), jnp.int32),
            pltpu.SemaphoreType.DMA,
        ],
    )
    def sc_gather(table_hbm, idx_hbm, out_hbm, idx_v, rows_v, sem):
        wid = lax.axis_index("s") * _NC + lax.axis_index("c")
        base = wid * _B_PER_W
        pltpu.sync_copy(idx_hbm.at[pl.ds(base, _B_PER_W)], idx_v)
        for v in range(_B_PER_W // 16):
            idx_v[pl.ds(v * 16, 16)] = idx_v[pl.ds(v * 16, 16)] & (_NQ - 1)
        pltpu.async_copy(table_hbm.at[idx_v], rows_v, sem).wait()
        pltpu.sync_copy(rows_v, out_hbm.at[pl.ds(base, _B_PER_W)])

    return sc_gather


def _mlp_body(g_ref, word_ref, prev_ref, wh1_ref, wh2_ref, bh_ref, wo_ref,
              bo_ref, out_ref):
    word = word_ref[...]
    u = lax.shift_right_logical(word, 18)
    right = u >= 2
    high = (u & 1) == 1
    g = g_ref[...]
    h16 = jnp.where(right, g[:, EMB:2 * EMB], g[:, 0:EMB])
    bits = jnp.where(high, h16 & _MASK_HI, lax.shift_left(h16, 16))
    x1 = lax.bitcast_convert_type(bits, jnp.float32)
    h = jnp.dot(x1, wh1_ref[...], preferred_element_type=jnp.float32)
    h = h + jnp.dot(prev_ref[...], wh2_ref[...], preferred_element_type=jnp.float32)
    h = h + bh_ref[...]
    y = h * jax.nn.sigmoid(h)
    logits = jnp.dot(y, wo_ref[...], preferred_element_type=jnp.float32) + bo_ref[...]
    m = jnp.max(logits, axis=-1, keepdims=True)
    e = jnp.exp(logits - m)
    out_ref[...] = e / jnp.sum(e, axis=-1, keepdims=True)


_BB = 2048


def _mlp(g, word2d, prev, wh1, wh2, bh, wo, bo):
    return pl.pallas_call(
        _mlp_body,
        grid=(B // _BB,),
        in_specs=[
            pl.BlockSpec((_BB, 2 * EMB), lambda i: (i, 0)),
            pl.BlockSpec((_BB, 1), lambda i: (i, 0)),
            pl.BlockSpec((_BB, NUM_LABELS), lambda i: (i, 0)),
            pl.BlockSpec((EMB, HID), lambda i: (0, 0)),
            pl.BlockSpec((NUM_LABELS, HID), lambda i: (0, 0)),
            pl.BlockSpec((1, HID), lambda i: (0, 0)),
            pl.BlockSpec((HID, NUM_LABELS), lambda i: (0, 0)),
            pl.BlockSpec((1, NUM_LABELS), lambda i: (0, 0)),
        ],
        out_specs=pl.BlockSpec((_BB, NUM_LABELS), lambda i: (i, 0)),
        out_shape=jax.ShapeDtypeStruct((B, NUM_LABELS), jnp.float32),
    )(g, word2d, prev, wh1, wh2, bh, wo, bo)


@jax.jit
def kernel(word, prev_label, emb_table, W_h, b_h, W_o, b_o):
    ident = jnp.eye(EMB, dtype=jnp.float32)
    packed = _pack(emb_table.T, ident)
    g = _make_sc_gather()(packed, word)
    return _mlp(
        g,
        word.reshape(B, 1),
        prev_label,
        W_h[:EMB],
        W_h[EMB:],
        b_h.reshape(1, HID),
        W_o,
        b_o.reshape(1, NUM_LABELS),
    )


# speedup vs baseline: 4.1023x; 1.2659x over previous
import functools

import jax
import jax.numpy as jnp
from jax import lax
from jax.experimental import pallas as pl
from jax.experimental.pallas import tpu as pltpu
from jax.experimental.pallas import tpu_sc as plsc

B = 16384
VOCAB = 1000000
EMB = 64
NUM_LABELS = 5
HID = 200

_NQ = 1 << 18
_NC = 2
_NS = 16
_NW = _NC * _NS
_B_PER_W = B // _NW

_BP = 16384
_MASK_HI = -65536


def _pack_body(t0_ref, t1_ref, t2_ref, t3_ref, ident_ref, out_ref):
    ident = ident_ref[...].astype(jnp.bfloat16)
    dn = (((0,), (0,)), ((), ()))

    def bits(t_ref):
        x = lax.dot_general(t_ref[...].astype(jnp.bfloat16), ident, dn,
                            preferred_element_type=jnp.float32)
        return lax.bitcast_convert_type(x, jnp.int32)

    b0, b1, b2, b3 = bits(t0_ref), bits(t1_ref), bits(t2_ref), bits(t3_ref)
    out_ref[:, 0:EMB] = lax.shift_right_logical(b0, 16) | (b1 & _MASK_HI)
    out_ref[:, EMB:2 * EMB] = lax.shift_right_logical(b2, 16) | (b3 & _MASK_HI)


def _pack(tT, ident):
    nb = _NQ // _BP
    last = (VOCAB - 1) // _BP

    def qmap(k):
        off = k * nb
        return lambda i, _off=off, _last=last: (0, jnp.minimum(i + _off, _last))

    return pl.pallas_call(
        _pack_body,
        grid=(nb,),
        in_specs=[
            pl.BlockSpec((EMB, _BP), qmap(0)),
            pl.BlockSpec((EMB, _BP), qmap(1)),
            pl.BlockSpec((EMB, _BP), qmap(2)),
            pl.BlockSpec((EMB, _BP), qmap(3)),
            pl.BlockSpec((EMB, EMB), lambda i: (0, 0)),
        ],
        out_specs=pl.BlockSpec((_BP, 2 * EMB), lambda i: (i, 0)),
        out_shape=jax.ShapeDtypeStruct((_NQ, 2 * EMB), jnp.int32),
    )(tT, tT, tT, tT, ident)


@functools.cache
def _make_sc_gather():
    mesh = plsc.VectorSubcoreMesh(core_axis_name="c", subcore_axis_name="s")

    @functools.partial(
        pl.kernel,
        mesh=mesh,
        out_type=jax.ShapeDtypeStruct((B, 2 * EMB), jnp.int32),
        scratch_types=[
            pltpu.VMEM((_B_PER_W,), jnp.int32),
            pltpu.VMEM((_B_PER_W, 2 * EMB), jnp.int32),
            pltpu.SemaphoreType.DMA,
        ],
    )
    def sc_gather(table_hbm, idx_hbm, out_hbm, idx_v, rows_v, sem):
        wid = lax.axis_index("s") * _NC + lax.axis_index("c")
        base = wid * _B_PER_W
        pltpu.sync_copy(idx_hbm.at[pl.ds(base, _B_PER_W)], idx_v)
        for v in range(_B_PER_W // 16):
            idx_v[pl.ds(v * 16, 16)] = idx_v[pl.ds(v * 16, 16)] & (_NQ - 1)
        pltpu.async_copy(table_hbm.at[idx_v], rows_v, sem).wait()
        pltpu.sync_copy(rows_v, out_hbm.at[pl.ds(base, _B_PER_W)])

    return sc_gather


def _mlp_body(g_ref, word_ref, prev_ref, wh1_ref, wh2_ref, bh_ref, wo_ref,
              bo_ref, out_ref):
    word = word_ref[...]
    u = lax.shift_right_logical(word, 18)
    right = u >= 2
    high = (u & 1) == 1
    g = g_ref[...]
    h16 = jnp.where(right, g[:, EMB:2 * EMB], g[:, 0:EMB])
    bits = jnp.where(high, h16 & _MASK_HI, lax.shift_left(h16, 16))
    x1 = lax.bitcast_convert_type(bits, jnp.float32)
    h = jnp.dot(x1, wh1_ref[...], preferred_element_type=jnp.float32)
    h = h + jnp.dot(prev_ref[...], wh2_ref[...], preferred_element_type=jnp.float32)
    h = h + bh_ref[...]
    y = h * jax.nn.sigmoid(h)
    logits = jnp.dot(y, wo_ref[...], preferred_element_type=jnp.float32) + bo_ref[...]
    m = jnp.max(logits, axis=-1, keepdims=True)
    e = jnp.exp(logits - m)
    out_ref[...] = e / jnp.sum(e, axis=-1, keepdims=True)


_BB = 2048


def _mlp(g, word2d, prev, wh1, wh2, bh, wo, bo):
    return pl.pallas_call(
        _mlp_body,
        grid=(B // _BB,),
        in_specs=[
            pl.BlockSpec((_BB, 2 * EMB), lambda i: (i, 0)),
            pl.BlockSpec((_BB, 1), lambda i: (i, 0)),
            pl.BlockSpec((_BB, NUM_LABELS), lambda i: (i, 0)),
            pl.BlockSpec((EMB, HID), lambda i: (0, 0)),
            pl.BlockSpec((NUM_LABELS, HID), lambda i: (0, 0)),
            pl.BlockSpec((1, HID), lambda i: (0, 0)),
            pl.BlockSpec((HID, NUM_LABELS), lambda i: (0, 0)),
            pl.BlockSpec((1, NUM_LABELS), lambda i: (0, 0)),
        ],
        out_specs=pl.BlockSpec((_BB, NUM_LABELS), lambda i: (i, 0)),
        out_shape=jax.ShapeDtypeStruct((B, NUM_LABELS), jnp.float32),
    )(g, word2d, prev, wh1, wh2, bh, wo, bo)


@jax.jit
def kernel(word, prev_label, emb_table, W_h, b_h, W_o, b_o):
    ident = jnp.eye(EMB, dtype=jnp.float32)
    packed = _pack(emb_table.T, ident)
    g = _make_sc_gather()(packed, word)
    return _mlp(
        g,
        word.reshape(B, 1),
        prev_label,
        W_h[:EMB],
        W_h[EMB:],
        b_h.reshape(1, HID),
        W_o,
        b_o.reshape(1, NUM_LABELS),
    )
